# baseline (speedup 1.0000x reference)
import jax
import jax.numpy as jnp
from jax.experimental import pallas as pl
from jax.experimental.pallas import tpu as pltpu


def _layers_kernel(GA, BA, C, x_ref, w1t_ref, adj_ref, b1_ref, w2_ref,
                   b2_ref, z_ref, s1t_ref, s2_ref):
    t = pl.program_id(0)

    @pl.when(t == 0)
    def _init():
        s1t_ref[...] = jax.lax.dot_general(
            w1t_ref[...], x_ref[...], (((1,), (1,)), ((), ())),
            preferred_element_type=jnp.float32)

    @pl.when(t < GA)
    def _phase0():
        h = jax.lax.dot_general(
            adj_ref[...], s1t_ref[...], (((1,), (1,)), ((), ())),
            preferred_element_type=jnp.float32) + b1_ref[...]
        h = jnp.maximum(h, 0.0)
        s2_ref[pl.ds(t * BA, BA), :] = jnp.dot(
            h, w2_ref[...], preferred_element_type=jnp.float32)

    @pl.when(t >= GA)
    def _phase1():
        z_ref[...] = jnp.dot(
            adj_ref[...], s2_ref[...],
            preferred_element_type=jnp.float32) + b2_ref[...]


def _decode_kernel(zi_ref, z_ref, o_ref):
    zz = jax.lax.dot_general(
        zi_ref[...], z_ref[...], (((1,), (1,)), ((), ())),
        preferred_element_type=jnp.float32)
    o_ref[...] = jax.nn.sigmoid(zz)


def kernel(x, adj, W1, b1, W2, b2):
    N, F = x.shape
    H = W1.shape[1]
    C = W2.shape[1]
    b1r = b1.reshape(1, H)
    b2r = b2.reshape(1, C)

    BA = 400
    GA = N // BA

    def adj_map(t):
        return (t % GA, 0)

    def z_map(t):
        return (jnp.where(t < GA, 0, t - GA), 0)

    layers = lambda *refs: _layers_kernel(GA, BA, C, *refs)
    z = pl.pallas_call(
        layers,
        grid=(2 * GA,),
        in_specs=[
            pl.BlockSpec((N, F), lambda t: (0, 0)),
            pl.BlockSpec((H, F), lambda t: (0, 0)),
            pl.BlockSpec((BA, N), adj_map),
            pl.BlockSpec((1, H), lambda t: (0, 0)),
            pl.BlockSpec((H, C), lambda t: (0, 0)),
            pl.BlockSpec((1, C), lambda t: (0, 0)),
        ],
        out_specs=pl.BlockSpec((BA, C), z_map),
        out_shape=jax.ShapeDtypeStruct((N, C), jnp.float32),
        scratch_shapes=[
            pltpu.VMEM((H, N), jnp.float32),
            pltpu.VMEM((N, C), jnp.float32),
        ],
    )(x, W1.T, adj, b1r, W2, b2r)

    BD = 400
    GD = N // BD
    a_pred = pl.pallas_call(
        _decode_kernel,
        grid=(GD,),
        in_specs=[
            pl.BlockSpec((BD, C), lambda i: (i, 0)),
            pl.BlockSpec((N, C), lambda i: (0, 0)),
        ],
        out_specs=pl.BlockSpec((BD, N), lambda i: (i, 0)),
        out_shape=jax.ShapeDtypeStruct((N, N), jnp.float32),
    )(z, z)
    return a_pred
